# baseline (speedup 1.0000x reference)
import functools

import jax
import jax.numpy as jnp
from jax.experimental import pallas as pl
from jax.experimental.pallas import tpu as pltpu

N = 10000
D = 128
NK = 2
H = 128
OUT = 64
BLK = 1000


def _mlp_body(x_ref, x1_ref, x2_ref, w1_ref, b1_ref, w2_ref, b2_ref,
              enc_ref, out_ref):
    x = x_ref[...]
    nrm = jnp.sqrt(jnp.sum(x * x, axis=1, keepdims=True))
    out0 = x / jnp.maximum(nrm, 1e-12)
    x1 = x1_ref[...]
    x2 = x2_ref[...]
    h = (jnp.dot(out0, w1_ref[0:D, :], preferred_element_type=jnp.float32)
         + jnp.dot(x1, w1_ref[D:D + NK * D, :], preferred_element_type=jnp.float32)
         + jnp.dot(x2, w1_ref[D + NK * D:, :], preferred_element_type=jnp.float32)
         + b1_ref[...])
    h = jnp.maximum(h, 0.0)
    enc_ref[...] = jnp.dot(h, w2_ref[...], preferred_element_type=jnp.float32) + b2_ref[...]
    out_ref[:, 0:D] = out0
    out_ref[:, D:D + NK * D] = x1
    out_ref[:, D + NK * D:] = x2


@jax.jit
def _mlp(x, x1, x2, W1, b1, W2, b2):
    total = D + NK * D + NK * NK * D
    grid = N // BLK
    return pl.pallas_call(
        _mlp_body,
        grid=(grid,),
        in_specs=[
            pl.BlockSpec((BLK, D), lambda i: (i, 0)),
            pl.BlockSpec((BLK, NK * D), lambda i: (i, 0)),
            pl.BlockSpec((BLK, NK * NK * D), lambda i: (i, 0)),
            pl.BlockSpec((total, H), lambda i: (0, 0)),
            pl.BlockSpec((1, H), lambda i: (0, 0)),
            pl.BlockSpec((H, OUT), lambda i: (0, 0)),
            pl.BlockSpec((1, OUT), lambda i: (0, 0)),
        ],
        out_specs=[
            pl.BlockSpec((BLK, OUT), lambda i: (i, 0)),
            pl.BlockSpec((BLK, total), lambda i: (i, 0)),
        ],
        out_shape=[
            jax.ShapeDtypeStruct((N, OUT), jnp.float32),
            jax.ShapeDtypeStruct((N, total), jnp.float32),
        ],
    )(x, x1, x2, W1, b1.reshape(1, H), W2, b2.reshape(1, OUT))


def kernel(x, edge_index, K_vals, W1, b1, W2, b2):
    src = edge_index[0]
    dst = edge_index[1]
    g = jnp.take(x, src, axis=0)
    x1 = jnp.concatenate(
        [jax.ops.segment_sum(K_vals[k][:, None] * g, dst, num_segments=N)
         for k in range(NK)], axis=-1)
    g1 = jnp.take(x1, src, axis=0)
    x2 = jnp.concatenate(
        [jax.ops.segment_sum(K_vals[k][:, None] * g1, dst, num_segments=N)
         for k in range(NK)], axis=-1)
    enc, out = _mlp(x, x1, x2, W1, b1, W2, b2)
    return (enc, out)
